# baseline (device time: 87874 ns/iter reference)
import jax
import jax.numpy as jnp
from jax import lax
from jax.experimental import pallas as pl
from jax.experimental.pallas import tpu as pltpu

N_DEV = 4
SQ = 1024
SKV = 1024
H_PER = 8
DH = 128
DMODEL = 1024
BLK = 64
QCH = 256
CH = 128
SCALE = 0.08838834764831843


def _body(idx_ref, x_ref, wq_ref, k_ref, v_ref, wo_ref, out_ref,
          q_scr, ctx_scr, kb_scr, vb_scr, wo_scr,
          cw_send, cw_recv, ccw_send, ccw_recv,
          cw_ssem, cw_rsem, ccw_ssem, ccw_rsem):
    my = lax.axis_index("i")
    left = lax.rem(my + N_DEV - 1, N_DEV)
    right = lax.rem(my + 1, N_DEV)

    barrier_sem = pltpu.get_barrier_semaphore()
    pl.semaphore_signal(barrier_sem, inc=1, device_id=(left,),
                        device_id_type=pl.DeviceIdType.MESH)
    pl.semaphore_signal(barrier_sem, inc=1, device_id=(right,),
                        device_id_type=pl.DeviceIdType.MESH)
    pl.semaphore_wait(barrier_sem, 2)

    kb_scr[:, :] = k_ref[:, :].astype(jnp.bfloat16)
    vb_scr[:, :] = v_ref[:, :].astype(jnp.bfloat16)
    wo_scr[:, :] = wo_ref[:, :].astype(jnp.bfloat16)
    q_scr[:, :] = (jnp.dot(x_ref[:, :].astype(jnp.bfloat16),
                           wq_ref[:, :].astype(jnp.bfloat16),
                           preferred_element_type=jnp.float32)
                   * SCALE).astype(jnp.bfloat16)

    for c in range(SQ // QCH):
        kl = QCH * (c + 1)
        row_blk = (c * QCH + lax.broadcasted_iota(jnp.int32, (QCH, kl), 0)
                   ) // BLK
        col_blk = lax.broadcasted_iota(jnp.int32, (QCH, kl), 1) // BLK
        mask = col_blk <= row_blk
        for h in range(H_PER):
            q = q_scr[c * QCH:(c + 1) * QCH, h * DH:(h + 1) * DH]
            s = lax.dot_general(
                q, kb_scr[:kl, h * DH:(h + 1) * DH],
                dimension_numbers=(((1,), (1,)), ((), ())),
                preferred_element_type=jnp.float32)
            w = jnp.exp(jnp.where(mask, s, -1e9))
            d = jnp.sum(w, axis=-1, keepdims=True)
            ctx = jnp.dot(w.astype(jnp.bfloat16),
                          vb_scr[:kl, h * DH:(h + 1) * DH],
                          preferred_element_type=jnp.float32)
            ctx_scr[c * QCH:(c + 1) * QCH,
                    h * DH:(h + 1) * DH] = (ctx / d).astype(jnp.bfloat16)

    def cw_rows(c):
        return pl.ds(lax.rem(c + 2 * N_DEV, N_DEV) * CH, CH)

    def ccw_rows(c):
        return pl.ds(N_DEV * CH + lax.rem(c + 2 * N_DEV, N_DEV) * CH, CH)

    def out_chunk(rows):
        out_ref[rows, :] = jnp.dot(ctx_scr[rows, :], wo_scr[:, :],
                                   preferred_element_type=jnp.float32)

    def mk_step(k):
        cw_src = cw_send.at[k] if k <= 3 else cw_recv.at[k - 1]
        ccw_src = ccw_send.at[k] if k <= 3 else ccw_recv.at[k - 1]
        cw = pltpu.make_async_remote_copy(
            src_ref=cw_src, dst_ref=cw_recv.at[k],
            send_sem=cw_ssem.at[k], recv_sem=cw_rsem.at[k],
            device_id=(right,), device_id_type=pl.DeviceIdType.MESH)
        ccw = pltpu.make_async_remote_copy(
            src_ref=ccw_src, dst_ref=ccw_recv.at[k],
            send_sem=ccw_ssem.at[k], recv_sem=ccw_rsem.at[k],
            device_id=(left,), device_id_type=pl.DeviceIdType.MESH)
        return cw, ccw

    steps = [mk_step(k) for k in range(6)]

    def start(k):
        steps[k][0].start()
        steps[k][1].start()

    def wait(k):
        steps[k][0].wait()
        steps[k][1].wait()

    out_chunk(cw_rows(my))
    out_chunk(ccw_rows(my))
    cw_send[0, :, :] = out_ref[cw_rows(my), :].astype(jnp.bfloat16)
    ccw_send[0, :, :] = out_ref[ccw_rows(my), :].astype(jnp.bfloat16)
    start(0)

    for k in range(2):
        out_chunk(cw_rows(my - k - 1))
        out_chunk(ccw_rows(my + k + 1))
        wait(k)
        cw_send[k + 1, :, :] = (
            out_ref[cw_rows(my - k - 1), :]
            + cw_recv[k, :, :].astype(jnp.float32)).astype(jnp.bfloat16)
        ccw_send[k + 1, :, :] = (
            out_ref[ccw_rows(my + k + 1), :]
            + ccw_recv[k, :, :].astype(jnp.float32)).astype(jnp.bfloat16)
        start(k + 1)

    out_chunk(cw_rows(my + 1))
    out_chunk(ccw_rows(my - 1))
    wait(2)
    red = out_ref[cw_rows(my + 1), :] + cw_recv[2, :, :].astype(jnp.float32)
    out_ref[cw_rows(my + 1), :] = red
    cw_send[3, :, :] = red.astype(jnp.bfloat16)
    red = out_ref[ccw_rows(my - 1), :] + ccw_recv[2, :, :].astype(jnp.float32)
    out_ref[ccw_rows(my - 1), :] = red
    ccw_send[3, :, :] = red.astype(jnp.bfloat16)
    start(3)

    wait(3)
    start(4)
    out_ref[cw_rows(my), :] = cw_recv[3, :, :].astype(jnp.float32)
    out_ref[ccw_rows(my), :] = ccw_recv[3, :, :].astype(jnp.float32)
    wait(4)
    start(5)
    out_ref[cw_rows(my - 1), :] = cw_recv[4, :, :].astype(jnp.float32)
    out_ref[ccw_rows(my + 1), :] = ccw_recv[4, :, :].astype(jnp.float32)
    wait(5)
    out_ref[cw_rows(my - 2), :] = cw_recv[5, :, :].astype(jnp.float32)
    out_ref[ccw_rows(my + 2), :] = ccw_recv[5, :, :].astype(jnp.float32)


def kernel(x, Wq, K_ext, V_ext, Wo):
    i = lax.axis_index("i")
    x2 = x.reshape(SQ, DMODEL)
    k2 = K_ext.reshape(SKV, 32 * DH)
    v2 = V_ext.reshape(SKV, 32 * DH)
    idx = jnp.reshape(i, (1,)).astype(jnp.int32)

    full = lambda g, s: (0, 0)
    myblk = lambda g, s: (0, s[0])
    grid_spec = pltpu.PrefetchScalarGridSpec(
        num_scalar_prefetch=1,
        grid=(1,),
        in_specs=[
            pl.BlockSpec((SQ, DMODEL), full),
            pl.BlockSpec((DMODEL, DMODEL), full),
            pl.BlockSpec((SKV, DMODEL), myblk),
            pl.BlockSpec((SKV, DMODEL), myblk),
            pl.BlockSpec((DMODEL, DMODEL), full),
        ],
        out_specs=pl.BlockSpec((SQ, DMODEL), full),
        scratch_shapes=[
            pltpu.VMEM((SQ, DMODEL), jnp.bfloat16),
            pltpu.VMEM((SQ, DMODEL), jnp.bfloat16),
            pltpu.VMEM((SKV, DMODEL), jnp.bfloat16),
            pltpu.VMEM((SKV, DMODEL), jnp.bfloat16),
            pltpu.VMEM((DMODEL, DMODEL), jnp.bfloat16),
            pltpu.VMEM((4, CH, DMODEL), jnp.bfloat16),
            pltpu.VMEM((6, CH, DMODEL), jnp.bfloat16),
            pltpu.VMEM((4, CH, DMODEL), jnp.bfloat16),
            pltpu.VMEM((6, CH, DMODEL), jnp.bfloat16),
            pltpu.SemaphoreType.DMA((6,)),
            pltpu.SemaphoreType.DMA((6,)),
            pltpu.SemaphoreType.DMA((6,)),
            pltpu.SemaphoreType.DMA((6,)),
        ],
    )

    out = pl.pallas_call(
        _body,
        grid_spec=grid_spec,
        out_shape=jax.ShapeDtypeStruct((SQ, DMODEL), jnp.float32),
        compiler_params=pltpu.CompilerParams(collective_id=0),
    )(idx, x2, Wq, k2, v2, Wo)
    return out.reshape(1, SQ, DMODEL)


# device time: 52619 ns/iter; 1.6700x vs baseline; 1.6700x over previous
import jax
import jax.numpy as jnp
from jax import lax
from jax.experimental import pallas as pl
from jax.experimental.pallas import tpu as pltpu

N_DEV = 4
SQ = 1024
SKV = 1024
H_PER = 8
DH = 128
DMODEL = 1024
BLK = 64
QCH = 256
CH = 128
SCALE = 0.08838834764831843


def _body(x_ref, wq_ref, k_ref, v_ref, wo_ref, out_ref,
          q_scr, ctx_scr, wo_scr,
          cw_send, cw_recv, ccw_send, ccw_recv,
          cw_ssem, cw_rsem, ccw_ssem, ccw_rsem):
    my = lax.axis_index("i")
    left = lax.rem(my + N_DEV - 1, N_DEV)
    right = lax.rem(my + 1, N_DEV)

    barrier_sem = pltpu.get_barrier_semaphore()
    pl.semaphore_signal(barrier_sem, inc=1, device_id=(left,),
                        device_id_type=pl.DeviceIdType.MESH)
    pl.semaphore_signal(barrier_sem, inc=1, device_id=(right,),
                        device_id_type=pl.DeviceIdType.MESH)
    pl.semaphore_wait(barrier_sem, 2)

    wo_scr[:, :] = wo_ref[:, :].astype(jnp.bfloat16)
    q_scr[:, :] = (jnp.dot(x_ref[:, :].astype(jnp.bfloat16),
                           wq_ref[:, :].astype(jnp.bfloat16),
                           preferred_element_type=jnp.float32)
                   * SCALE).astype(jnp.bfloat16)

    for c in range(SQ // QCH):
        kl = QCH * (c + 1)
        row_blk = (c * QCH + lax.broadcasted_iota(jnp.int32, (QCH, kl), 0)
                   ) // BLK
        col_blk = lax.broadcasted_iota(jnp.int32, (QCH, kl), 1) // BLK
        mask = col_blk <= row_blk
        for h in range(H_PER):
            q = q_scr[c * QCH:(c + 1) * QCH, h * DH:(h + 1) * DH]
            s = lax.dot_general(
                q, k_ref[h, :kl, :],
                dimension_numbers=(((1,), (1,)), ((), ())),
                preferred_element_type=jnp.float32)
            w = jnp.exp(jnp.where(mask, s, -1e9))
            d = jnp.sum(w, axis=-1, keepdims=True)
            ctx = jnp.dot(w.astype(jnp.bfloat16), v_ref[h, :kl, :],
                          preferred_element_type=jnp.float32)
            ctx_scr[c * QCH:(c + 1) * QCH,
                    h * DH:(h + 1) * DH] = (ctx / d).astype(jnp.bfloat16)

    def cw_rows(c):
        return pl.ds(lax.rem(c + 2 * N_DEV, N_DEV) * CH, CH)

    def ccw_rows(c):
        return pl.ds(N_DEV * CH + lax.rem(c + 2 * N_DEV, N_DEV) * CH, CH)

    def out_chunk(rows):
        out_ref[rows, :] = jnp.dot(ctx_scr[rows, :], wo_scr[:, :],
                                   preferred_element_type=jnp.float32)

    def mk_step(k):
        cw_src = cw_send.at[k] if k <= 3 else cw_recv.at[k - 1]
        ccw_src = ccw_send.at[k] if k <= 3 else ccw_recv.at[k - 1]
        cw = pltpu.make_async_remote_copy(
            src_ref=cw_src, dst_ref=cw_recv.at[k],
            send_sem=cw_ssem.at[k], recv_sem=cw_rsem.at[k],
            device_id=(right,), device_id_type=pl.DeviceIdType.MESH)
        ccw = pltpu.make_async_remote_copy(
            src_ref=ccw_src, dst_ref=ccw_recv.at[k],
            send_sem=ccw_ssem.at[k], recv_sem=ccw_rsem.at[k],
            device_id=(left,), device_id_type=pl.DeviceIdType.MESH)
        return cw, ccw

    steps = [mk_step(k) for k in range(6)]

    def start(k):
        steps[k][0].start()
        steps[k][1].start()

    def wait(k):
        steps[k][0].wait()
        steps[k][1].wait()

    out_chunk(cw_rows(my))
    out_chunk(ccw_rows(my))
    cw_send[0, :, :] = out_ref[cw_rows(my), :].astype(jnp.bfloat16)
    ccw_send[0, :, :] = out_ref[ccw_rows(my), :].astype(jnp.bfloat16)
    start(0)

    for k in range(2):
        out_chunk(cw_rows(my - k - 1))
        out_chunk(ccw_rows(my + k + 1))
        wait(k)
        cw_send[k + 1, :, :] = (
            out_ref[cw_rows(my - k - 1), :]
            + cw_recv[k, :, :].astype(jnp.float32)).astype(jnp.bfloat16)
        ccw_send[k + 1, :, :] = (
            out_ref[ccw_rows(my + k + 1), :]
            + ccw_recv[k, :, :].astype(jnp.float32)).astype(jnp.bfloat16)
        start(k + 1)

    out_chunk(cw_rows(my + 1))
    out_chunk(ccw_rows(my - 1))
    wait(2)
    red = out_ref[cw_rows(my + 1), :] + cw_recv[2, :, :].astype(jnp.float32)
    out_ref[cw_rows(my + 1), :] = red
    cw_send[3, :, :] = red.astype(jnp.bfloat16)
    red = out_ref[ccw_rows(my - 1), :] + ccw_recv[2, :, :].astype(jnp.float32)
    out_ref[ccw_rows(my - 1), :] = red
    ccw_send[3, :, :] = red.astype(jnp.bfloat16)
    start(3)

    wait(3)
    start(4)
    out_ref[cw_rows(my), :] = cw_recv[3, :, :].astype(jnp.float32)
    out_ref[ccw_rows(my), :] = ccw_recv[3, :, :].astype(jnp.float32)
    wait(4)
    start(5)
    out_ref[cw_rows(my - 1), :] = cw_recv[4, :, :].astype(jnp.float32)
    out_ref[ccw_rows(my + 1), :] = ccw_recv[4, :, :].astype(jnp.float32)
    wait(5)
    out_ref[cw_rows(my - 2), :] = cw_recv[5, :, :].astype(jnp.float32)
    out_ref[ccw_rows(my + 2), :] = ccw_recv[5, :, :].astype(jnp.float32)


def kernel(x, Wq, K_ext, V_ext, Wo):
    i = lax.axis_index("i")
    x2 = x.reshape(SQ, DMODEL)
    k = lax.dynamic_slice_in_dim(
        K_ext.reshape(SKV, 32, DH), i * H_PER, H_PER, axis=1)
    v = lax.dynamic_slice_in_dim(
        V_ext.reshape(SKV, 32, DH), i * H_PER, H_PER, axis=1)
    kb = k.transpose(1, 0, 2).astype(jnp.bfloat16)
    vb = v.transpose(1, 0, 2).astype(jnp.bfloat16)

    out = pl.pallas_call(
        _body,
        out_shape=jax.ShapeDtypeStruct((SQ, DMODEL), jnp.float32),
        in_specs=[pl.BlockSpec(memory_space=pltpu.VMEM)] * 5,
        out_specs=pl.BlockSpec(memory_space=pltpu.VMEM),
        scratch_shapes=[
            pltpu.VMEM((SQ, DMODEL), jnp.bfloat16),
            pltpu.VMEM((SQ, DMODEL), jnp.bfloat16),
            pltpu.VMEM((DMODEL, DMODEL), jnp.bfloat16),
            pltpu.VMEM((4, CH, DMODEL), jnp.bfloat16),
            pltpu.VMEM((6, CH, DMODEL), jnp.bfloat16),
            pltpu.VMEM((4, CH, DMODEL), jnp.bfloat16),
            pltpu.VMEM((6, CH, DMODEL), jnp.bfloat16),
            pltpu.SemaphoreType.DMA((6,)),
            pltpu.SemaphoreType.DMA((6,)),
            pltpu.SemaphoreType.DMA((6,)),
            pltpu.SemaphoreType.DMA((6,)),
        ],
        compiler_params=pltpu.CompilerParams(collective_id=0),
    )(x2, Wq, kb, vb, Wo)
    return out.reshape(1, SQ, DMODEL)
